# baseline (speedup 1.0000x reference)
import dataclasses
import functools

import jax
import jax.numpy as jnp
from jax import lax
from jax.experimental import pallas as pl
from jax.experimental.pallas import tpu as pltpu
from jax.experimental.pallas import tpu_sc as plsc

N_TOKENS = 32768
N_EMBED = 768
NUM_EXPERTS = 8
TOP_K = 2

BT = 4096
CHUNKS = 1

NC, NS, L = 2, 16, 16
NW = NC * NS


def _logits_kernel(x_ref, wt_ref, b_ref, lt_ref):
    logits = jax.lax.dot_general(
        x_ref[...], wt_ref[...],
        dimension_numbers=(((1,), (0,)), ((), ())),
        preferred_element_type=jnp.float32,
    )
    lt_ref[...] = logits.T + b_ref[...]


def _tc_logits_chunk(x, wt, b2, chunk_tokens, chunk_idx):
    steps = chunk_tokens // BT
    off = chunk_idx * steps
    return pl.pallas_call(
        _logits_kernel,
        grid=(steps,),
        in_specs=[
            pl.BlockSpec((BT, N_EMBED), lambda i: (i + off, 0)),
            pl.BlockSpec((N_EMBED, NUM_EXPERTS), lambda i: (0, 0)),
            pl.BlockSpec((NUM_EXPERTS, 1), lambda i: (0, 0)),
        ],
        out_specs=pl.BlockSpec((NUM_EXPERTS, BT), lambda i: (0, i)),
        out_shape=jax.ShapeDtypeStruct((NUM_EXPERTS, chunk_tokens),
                                       jnp.float32),
    )(x, wt, b2)


def _sc_route(lt):
    n = lt.shape[1]
    bpw = n // NW
    mesh = plsc.VectorSubcoreMesh(core_axis_name="c", subcore_axis_name="s")
    cp = pltpu.CompilerParams()
    if "needs_layout_passes" in pltpu.CompilerParams.__dataclass_fields__:
        cp = dataclasses.replace(cp, needs_layout_passes=False)

    @functools.partial(
        pl.kernel,
        mesh=mesh,
        compiler_params=cp,
        out_type=[
            jax.ShapeDtypeStruct((n * NUM_EXPERTS,), jnp.float32),
            jax.ShapeDtypeStruct((n * TOP_K,), jnp.int32),
        ],
        scratch_types=(
            [pltpu.VMEM((bpw,), jnp.float32) for _ in range(NUM_EXPERTS)]
            + [pltpu.VMEM((bpw * NUM_EXPERTS,), jnp.float32),
               pltpu.VMEM((bpw * TOP_K,), jnp.int32),
               pltpu.SemaphoreType.DMA]
        ),
    )
    def route(lt_hbm, out_hbm, idx_hbm, *scratch):
        l_refs = scratch[:NUM_EXPERTS]
        o_ref, i_ref, sem = scratch[NUM_EXPERTS:]
        wid = lax.axis_index("s") * NC + lax.axis_index("c")
        base = wid * bpw
        copies = [
            pltpu.async_copy(lt_hbm.at[e, pl.ds(base, bpw)], l_refs[e], sem)
            for e in range(NUM_EXPERTS)
        ]
        for c in copies:
            c.wait()

        @pl.loop(0, bpw, step=L)
        def _(c):
            sl = pl.ds(c, L)
            lv = [l_refs[e][sl] for e in range(NUM_EXPERTS)]
            m1 = lv[0]
            i1 = jnp.zeros((L,), jnp.int32)
            m2 = jnp.full((L,), -jnp.inf, jnp.float32)
            i2 = jnp.zeros((L,), jnp.int32)
            for e in range(1, NUM_EXPERTS):
                gt1 = lv[e] > m1
                gt2 = lv[e] > m2
                nm2 = jnp.where(gt1, m1, jnp.where(gt2, lv[e], m2))
                ni2 = jnp.where(gt1, i1, jnp.where(gt2, e, i2))
                m1 = jnp.where(gt1, lv[e], m1)
                i1 = jnp.where(gt1, e, i1)
                m2, i2 = nm2, ni2
            e2 = jnp.exp(m2 - m1)
            p1 = 1.0 / (1.0 + e2)
            p2 = e2 * p1
            zero = jnp.zeros((L,), jnp.float32)
            t8 = (c + lax.iota(jnp.int32, L)) * NUM_EXPERTS
            t2 = (c + lax.iota(jnp.int32, L)) * TOP_K
            for e in range(NUM_EXPERTS):
                v = jnp.where(i1 == e, p1, jnp.where(i2 == e, p2, zero))
                plsc.store_scatter(o_ref, [t8 + e], v)
            plsc.store_scatter(i_ref, [t2], i1)
            plsc.store_scatter(i_ref, [t2 + 1], i2)

        out_copies = [
            pltpu.async_copy(
                o_ref, out_hbm.at[pl.ds(base * NUM_EXPERTS, bpw * NUM_EXPERTS)],
                sem),
            pltpu.async_copy(
                i_ref, idx_hbm.at[pl.ds(base * TOP_K, bpw * TOP_K)], sem),
        ]
        for c in out_copies:
            c.wait()

    return route(lt)


@jax.jit
def kernel(x, W, b):
    n_tokens = x.shape[0]
    chunk = n_tokens // CHUNKS
    wt = W.T
    b2 = b.reshape(NUM_EXPERTS, 1)
    outs, idxs = [], []
    for c in range(CHUNKS):
        lt_c = _tc_logits_chunk(x, wt, b2, chunk, c)
        o_c, i_c = _sc_route(lt_c)
        outs.append(o_c.reshape(chunk, NUM_EXPERTS))
        idxs.append(i_c.reshape(chunk, TOP_K))
    if CHUNKS == 1:
        return outs[0], idxs[0]
    return jnp.concatenate(outs, 0), jnp.concatenate(idxs, 0)


# speedup vs baseline: 2.8668x; 2.8668x over previous
import functools

import jax
import jax.numpy as jnp
from jax.experimental import pallas as pl

N_TOKENS = 32768
N_EMBED = 768
NUM_EXPERTS = 8
TOP_K = 2

BT = 4096


def _router_kernel(x_ref, wt_ref, b_ref, outt_ref, idxt_ref):
    logits = jax.lax.dot_general(
        x_ref[...], wt_ref[...],
        dimension_numbers=(((1,), (0,)), ((), ())),
        preferred_element_type=jnp.float32,
    )
    lt = logits.T + b_ref[...]

    se = jax.lax.broadcasted_iota(jnp.int32, lt.shape, 0).astype(jnp.float32)
    m1 = jnp.max(lt, axis=0, keepdims=True)
    i1 = jnp.min(jnp.where(lt == m1, se, 8.0), axis=0, keepdims=True)
    masked = jnp.where(se == i1, -jnp.inf, lt)
    m2 = jnp.max(masked, axis=0, keepdims=True)
    i2 = jnp.min(jnp.where(masked == m2, se, 8.0), axis=0, keepdims=True)

    e2 = jnp.exp(m2 - m1)
    p1 = 1.0 / (1.0 + e2)
    p2 = e2 * p1

    outt_ref[...] = jnp.where(se == i1, p1, jnp.where(se == i2, p2, 0.0))
    idxt_ref[...] = jnp.concatenate([i1, i2], axis=0).astype(jnp.int32)


@functools.partial(jax.jit, static_argnames=())
def kernel(x, W, b):
    n_tokens = x.shape[0]
    grid = (n_tokens // BT,)
    wt = W.T
    b2 = b.reshape(NUM_EXPERTS, 1)
    outt, idxt = pl.pallas_call(
        _router_kernel,
        grid=grid,
        in_specs=[
            pl.BlockSpec((BT, N_EMBED), lambda i: (i, 0)),
            pl.BlockSpec((N_EMBED, NUM_EXPERTS), lambda i: (0, 0)),
            pl.BlockSpec((NUM_EXPERTS, 1), lambda i: (0, 0)),
        ],
        out_specs=[
            pl.BlockSpec((NUM_EXPERTS, BT), lambda i: (0, i)),
            pl.BlockSpec((TOP_K, BT), lambda i: (0, i)),
        ],
        out_shape=[
            jax.ShapeDtypeStruct((NUM_EXPERTS, n_tokens), jnp.float32),
            jax.ShapeDtypeStruct((TOP_K, n_tokens), jnp.int32),
        ],
    )(x, wt, b2)
    return outt.T, idxt.T
